# Initial kernel scaffold; baseline (speedup 1.0000x reference)
import jax
import jax.numpy as jnp
from jax.experimental import pallas as pl


def kernel(wav):
    raise NotImplementedError("write your pallas kernel here")


# speedup vs baseline: 5.0688x; 5.0688x over previous
import functools

import jax
import jax.numpy as jnp
from jax import lax
from jax.experimental import pallas as pl
from jax.experimental.pallas import tpu as pltpu
from jax.experimental.pallas import tpu_sc as plsc

DECIM = 3
LANES = 16

B = 16
N_IN = 480000
N_OUT = 160000

NUM_CORES = 2
NUM_SUBCORES = 16
NUM_WORKERS = NUM_CORES * NUM_SUBCORES

HALVES = NUM_WORKERS // B
OUT_PER_WORKER = N_OUT // HALVES
IN_PER_WORKER = OUT_PER_WORKER * DECIM

NO_CHUNK = 16000
NI_CHUNK = NO_CHUNK * DECIM
NUM_CHUNKS = OUT_PER_WORKER // NO_CHUNK


def _body(wav_hbm, out_hbm, in_v, out_v):
    wid = lax.axis_index("s") * NUM_CORES + lax.axis_index("c")
    row = wid // HALVES
    half = wid % HALVES
    idx0 = lax.iota(jnp.int32, LANES) * DECIM

    def chunk_body(k, _):
        in_start = half * IN_PER_WORKER + k * NI_CHUNK
        out_start = half * OUT_PER_WORKER + k * NO_CHUNK
        pltpu.sync_copy(wav_hbm.at[row, pl.ds(in_start, NI_CHUNK)], in_v)

        def gather_body(j, _):
            idx = idx0 + j * (LANES * DECIM)
            v = plsc.load_gather(in_v, [idx])
            out_v[pl.ds(j * LANES, LANES)] = v
            return 0

        lax.fori_loop(0, NO_CHUNK // LANES, gather_body, 0)
        pltpu.sync_copy(out_v, out_hbm.at[row, pl.ds(out_start, NO_CHUNK)])
        return 0

    lax.fori_loop(0, NUM_CHUNKS, chunk_body, 0)


@jax.jit
def kernel(wav):
    wav = wav.reshape(wav.shape[0], -1)
    assert wav.shape == (B, N_IN), wav.shape
    mesh = plsc.VectorSubcoreMesh(core_axis_name="c", subcore_axis_name="s")
    run = functools.partial(
        pl.kernel,
        mesh=mesh,
        out_type=jax.ShapeDtypeStruct((B, N_OUT), jnp.float32),
        scratch_types=[
            pltpu.VMEM((NI_CHUNK,), jnp.float32),
            pltpu.VMEM((NO_CHUNK,), jnp.float32),
        ],
        compiler_params=pltpu.CompilerParams(needs_layout_passes=False),
    )(_body)
    return run(wav)


# speedup vs baseline: 8.1737x; 1.6126x over previous
import functools

import jax
import jax.numpy as jnp
from jax import lax
from jax.experimental import pallas as pl
from jax.experimental.pallas import tpu as pltpu
from jax.experimental.pallas import tpu_sc as plsc

DECIM = 3
LANES = 16

B = 16
N_IN = 480000
N_OUT = 160000

NUM_CORES = 2
NUM_SUBCORES = 16
NUM_WORKERS = NUM_CORES * NUM_SUBCORES

HALVES = NUM_WORKERS // B
OUT_PER_WORKER = N_OUT // HALVES
IN_PER_WORKER = OUT_PER_WORKER * DECIM

NO_CHUNK = 16000
NI_CHUNK = NO_CHUNK * DECIM
NUM_CHUNKS = OUT_PER_WORKER // NO_CHUNK
UNROLL = 8


def _body(wav_hbm, out_hbm, in_buf0, in_buf1, out_buf0, out_buf1, sem_in0,
          sem_in1, sem_out0, sem_out1):
    wid = lax.axis_index("s") * NUM_CORES + lax.axis_index("c")
    row = wid // HALVES
    half = wid % HALVES
    idx0 = lax.iota(jnp.int32, LANES) * DECIM
    in_bufs = (in_buf0, in_buf1)
    out_bufs = (out_buf0, out_buf1)
    sems_in = (sem_in0, sem_in1)
    sems_out = (sem_out0, sem_out1)

    def in_copy(k, slot):
        src = wav_hbm.at[row, pl.ds(half * IN_PER_WORKER + k * NI_CHUNK,
                                    NI_CHUNK)]
        return pltpu.make_async_copy(src, in_bufs[slot], sems_in[slot])

    def out_copy(k, slot):
        dst = out_hbm.at[row, pl.ds(half * OUT_PER_WORKER + k * NO_CHUNK,
                                    NO_CHUNK)]
        return pltpu.make_async_copy(out_bufs[slot], dst, sems_out[slot])

    pending_out = [None, None]
    in_copy(0, 0).start()
    for k in range(NUM_CHUNKS):
        slot = k % 2
        if k + 1 < NUM_CHUNKS:
            in_copy(k + 1, 1 - slot).start()
        in_copy(k, slot).wait()
        if pending_out[slot] is not None:
            pending_out[slot].wait()
        in_ref = in_bufs[slot]
        out_ref = out_bufs[slot]

        @plsc.parallel_loop(0, NO_CHUNK // LANES, unroll=UNROLL)
        def _(j):
            idx = idx0 + j * (LANES * DECIM)
            out_ref[pl.ds(j * LANES, LANES)] = plsc.load_gather(in_ref, [idx])

        oc = out_copy(k, slot)
        oc.start()
        pending_out[slot] = oc
    for oc in pending_out:
        oc.wait()


@jax.jit
def kernel(wav):
    wav = wav.reshape(wav.shape[0], -1)
    assert wav.shape == (B, N_IN), wav.shape
    mesh = plsc.VectorSubcoreMesh(core_axis_name="c", subcore_axis_name="s")
    run = functools.partial(
        pl.kernel,
        mesh=mesh,
        out_type=jax.ShapeDtypeStruct((B, N_OUT), jnp.float32),
        scratch_types=[
            pltpu.VMEM((NI_CHUNK,), jnp.float32),
            pltpu.VMEM((NI_CHUNK,), jnp.float32),
            pltpu.VMEM((NO_CHUNK,), jnp.float32),
            pltpu.VMEM((NO_CHUNK,), jnp.float32),
            pltpu.SemaphoreType.DMA,
            pltpu.SemaphoreType.DMA,
            pltpu.SemaphoreType.DMA,
            pltpu.SemaphoreType.DMA,
        ],
        compiler_params=pltpu.CompilerParams(needs_layout_passes=False),
    )(_body)
    return run(wav)
